# baseline (speedup 1.0000x reference)
import functools

import jax
import jax.numpy as jnp
from jax import lax
from jax.experimental import pallas as pl
from jax.experimental.pallas import tpu as pltpu
from jax.experimental.pallas import tpu_sc as plsc

B = 16384
K = 32
NC = 2
NS = 16
NW = NC * NS
BPW = B // NW
RB = 16
NG = BPW // RB
NE = BPW * K
CH = 128
NCH = NE // CH
CPW = 8
NWAVE = NCH // CPW
EPW = CH * CPW

UROWS, VROWS = 1000000, 100000
UMAIN = (UROWS // 128) * 128
VMAIN = (VROWS // 128) * 128
USTRIDE = UMAIN + 128
VSTRIDE = VMAIN + 128
UTILES = UMAIN // 128
VTILES = VMAIN // 128
W = 39
WW = W * 128
UPER = (UTILES + 7) // 8
VPER = (VTILES + 7) // 8
UNCH = (UPER + W - 1) // W
VNCH = (VPER + W - 1) // W

_mesh = plsc.VectorSubcoreMesh(core_axis_name="c", subcore_axis_name="s")


@functools.partial(
    pl.kernel,
    mesh=_mesh,
    out_type=[
        jax.ShapeDtypeStruct((K * USTRIDE,), jnp.float32),
        jax.ShapeDtypeStruct((K * VSTRIDE,), jnp.float32),
    ],
    compiler_params=pltpu.CompilerParams(
        needs_layout_passes=False, use_tc_tiling_on_sc=True),
    scratch_types=[
        pltpu.VMEM((8, WW), jnp.float32),
        pltpu.VMEM((8, WW), jnp.float32),
        pltpu.VMEM((8, WW), jnp.float32),
        pltpu.SemaphoreType.DMA,
        pltpu.SemaphoreType.DMA,
    ],
)
def _relayout(uembT_hbm, vembT_hbm, utail_hbm, vtail_hbm,
              uflat_hbm, vflat_hbm, bufa, bufb, bufc, sem_in, sem_out):
    wid = lax.axis_index("s") * NC + lax.axis_index("c")
    g = wid // 8
    si = wid % 8
    bufs = (bufa, bufb, bufc)

    def chunk_col(start, last, j):
        return jnp.minimum(start + j * W, last)

    def fire_in(table, start, last, j, buf):
        c0 = chunk_col(start, last, j)
        pltpu.async_copy(
            table.at[pl.ds(pl.multiple_of(g * 8, 8), 8),
                     pl.ds(pl.multiple_of(c0 * 128, 128), WW)], buf, sem_in)

    def fire_outs(flat, stride, start, last, j, buf):
        c0 = chunk_col(start, last, j)
        for s in range(8):
            k = g * 8 + s
            pltpu.async_copy(
                buf.at[s, pl.ds(0, WW)],
                flat.at[pl.ds(pl.multiple_of(k * stride + c0 * 128, 128), WW)],
                sem_out)

    def wait_in():
        pltpu.make_async_copy(uembT_hbm.at[pl.ds(0, 8), pl.ds(0, WW)],
                              bufa, sem_in).wait()

    def drain_out():
        pltpu.make_async_copy(uembT_hbm.at[pl.ds(0, 8), pl.ds(0, WW)],
                              bufa, sem_out).wait()

    ustart = si * UPER
    ulast = jnp.minimum(ustart + UPER, UTILES) - W

    fire_in(uembT_hbm, ustart, ulast, 0, bufs[0])
    fire_in(uembT_hbm, ustart, ulast, 1, bufs[1])

    def u_triple(t, _):
        for i in range(3):
            j = t * 3 + i

            @pl.when(j + 2 < UNCH)
            def _():
                @pl.when(j >= 1)
                def _():
                    drain_out()
                fire_in(uembT_hbm, ustart, ulast, j + 2, bufs[(i + 2) % 3])

            wait_in()
            fire_outs(uflat_hbm, USTRIDE, ustart, ulast, j, bufs[i])
        return 0

    lax.fori_loop(0, UNCH // 3, u_triple, 0)
    for j in range((UNCH // 3) * 3, UNCH):
        wait_in()
        fire_outs(uflat_hbm, USTRIDE, ustart, ulast, j, bufs[j % 3])
    drain_out()
    drain_out()
    drain_out()

    vstart = si * VPER
    vlast = jnp.minimum(vstart + VPER, VTILES) - W

    fire_in(vembT_hbm, vstart, vlast, 0, bufs[0])
    fire_in(vembT_hbm, vstart, vlast, 1, bufs[1])
    for j in range(VNCH):
        if j + 2 < VNCH:
            if j >= 1:
                drain_out()
            fire_in(vembT_hbm, vstart, vlast, j + 2, bufs[(j + 2) % 3])
        wait_in()
        fire_outs(vflat_hbm, VSTRIDE, vstart, vlast, j, bufs[j % 3])
    for _ in range(VNCH - max(0, VNCH - 3)):
        drain_out()

    cp1 = pltpu.async_copy(
        utail_hbm.at[pl.ds(wid * 128, 128)],
        uflat_hbm.at[pl.ds(wid * USTRIDE + UMAIN, 128)], sem_in)
    cp2 = pltpu.async_copy(
        vtail_hbm.at[pl.ds(wid * 128, 128)],
        vflat_hbm.at[pl.ds(wid * VSTRIDE + VMAIN, 128)], sem_in)
    cp1.wait()
    cp2.wait()


@functools.partial(
    pl.kernel,
    mesh=_mesh,
    out_type=jax.ShapeDtypeStruct((B,), jnp.float32),
    compiler_params=pltpu.CompilerParams(
        needs_layout_passes=False, use_tc_tiling_on_sc=False),
    scratch_types=[
        pltpu.VMEM((BPW,), jnp.int32),
        pltpu.VMEM((BPW,), jnp.int32),
        pltpu.VMEM((NE,), jnp.int32),
        pltpu.VMEM((NE,), jnp.int32),
        pltpu.VMEM((NE,), jnp.float32),
        pltpu.VMEM((NE,), jnp.float32),
        pltpu.VMEM((BPW,), jnp.float32),
        pltpu.SemaphoreType.DMA,
    ],
)
def _lookup(uidx_hbm, vidx_hbm, uflat_hbm, vflat_hbm, out_hbm,
            uidx, vidx, uoff, voff, uel, vel, outv, sem):
    wid = lax.axis_index("s") * NC + lax.axis_index("c")
    base = wid * BPW
    pltpu.sync_copy(uidx_hbm.at[pl.ds(base, BPW)], uidx)
    pltpu.sync_copy(vidx_hbm.at[pl.ds(base, BPW)], vidx)

    def offsets(g, _):
        ru = uidx[pl.ds(g * RB, RB)]
        rv = vidx[pl.ds(g * RB, RB)]
        for k in range(K):
            uoff[pl.ds(k * BPW + g * RB, RB)] = ru + k * USTRIDE
            voff[pl.ds(k * BPW + g * RB, RB)] = rv + k * VSTRIDE
        return 0

    lax.fori_loop(0, NG, offsets, 0)

    def wave(w, _):
        for c0 in range(CPW):
            c = w * CPW + c0
            pltpu.async_copy(uflat_hbm.at[uoff.at[pl.ds(c * CH, CH)]],
                             uel.at[pl.ds(c * CH, CH)], sem)
            pltpu.async_copy(vflat_hbm.at[voff.at[pl.ds(c * CH, CH)]],
                             vel.at[pl.ds(c * CH, CH)], sem)

        @pl.when(w > 0)
        def _drain_prev():
            pltpu.make_async_copy(uflat_hbm.at[pl.ds(0, EPW)],
                                  uel.at[pl.ds(0, EPW)], sem).wait()
            pltpu.make_async_copy(vflat_hbm.at[pl.ds(0, EPW)],
                                  vel.at[pl.ds(0, EPW)], sem).wait()
        return 0

    lax.fori_loop(0, NWAVE, wave, 0)
    pltpu.make_async_copy(uflat_hbm.at[pl.ds(0, EPW)],
                          uel.at[pl.ds(0, EPW)], sem).wait()
    pltpu.make_async_copy(vflat_hbm.at[pl.ds(0, EPW)],
                          vel.at[pl.ds(0, EPW)], sem).wait()

    def block(b, _):
        acc = jnp.zeros((RB,), jnp.float32)
        for k in range(K):
            acc = acc + (uel[pl.ds(k * BPW + b * RB, RB)]
                         * vel[pl.ds(k * BPW + b * RB, RB)])
        outv[pl.ds(b * RB, RB)] = acc
        return 0

    lax.fori_loop(0, NG, block, 0)
    pltpu.sync_copy(outv, out_hbm.at[pl.ds(base, BPW)])


def kernel(user_index, item_index, Uemb, Vemb):
    utail = jnp.reshape(
        jnp.pad(Uemb[UMAIN:], ((0, 128 - (UROWS - UMAIN)), (0, 0))).T, (K * 128,))
    vtail = jnp.reshape(
        jnp.pad(Vemb[VMAIN:], ((0, 128 - (VROWS - VMAIN)), (0, 0))).T, (K * 128,))
    uflat, vflat = _relayout(Uemb.T, Vemb.T, utail, vtail)
    return _lookup(user_index.astype(jnp.int32), item_index.astype(jnp.int32),
                   uflat, vflat)


# speedup vs baseline: 1.0235x; 1.0235x over previous
import functools

import jax
import jax.numpy as jnp
from jax import lax
from jax.experimental import pallas as pl
from jax.experimental.pallas import tpu as pltpu
from jax.experimental.pallas import tpu_sc as plsc

B = 16384
K = 32
NC = 2
NS = 16
NW = NC * NS
BPW = B // NW
RB = 16
NG = BPW // RB
NE = BPW * K
CH = 128
NCH = NE // CH
CPW = 8
NWAVE = NCH // CPW
EPW = CH * CPW

UROWS, VROWS = 1000000, 100000
UMAIN = (UROWS // 128) * 128
VMAIN = (VROWS // 128) * 128
USTRIDE = UMAIN + 128
VSTRIDE = VMAIN + 128
UTILES = UMAIN // 128
VTILES = VMAIN // 128
W = 26
WW = W * 128
UPER = (UTILES + 7) // 8
VPER = (VTILES + 7) // 8
UNCH = (UPER + W - 1) // W
VNCH = (VPER + W - 1) // W

_mesh = plsc.VectorSubcoreMesh(core_axis_name="c", subcore_axis_name="s")


@functools.partial(
    pl.kernel,
    mesh=_mesh,
    out_type=[
        jax.ShapeDtypeStruct((K * USTRIDE,), jnp.float32),
        jax.ShapeDtypeStruct((K * VSTRIDE,), jnp.float32),
    ],
    compiler_params=pltpu.CompilerParams(
        needs_layout_passes=False, use_tc_tiling_on_sc=True),
    scratch_types=[
        pltpu.VMEM((8, WW), jnp.float32),
        pltpu.VMEM((8, WW), jnp.float32),
        pltpu.VMEM((8, WW), jnp.float32),
        pltpu.SemaphoreType.DMA,
        pltpu.SemaphoreType.DMA,
    ],
)
def _relayout(uembT_hbm, vembT_hbm, utail_hbm, vtail_hbm,
              uflat_hbm, vflat_hbm, bufa, bufb, bufc, sem_in, sem_out):
    wid = lax.axis_index("s") * NC + lax.axis_index("c")
    g = wid // 8
    si = wid % 8
    bufs = (bufa, bufb, bufc)

    def chunk_col(start, last, j):
        return jnp.minimum(start + j * W, last)

    def fire_in(table, start, last, j, buf):
        c0 = chunk_col(start, last, j)
        pltpu.async_copy(
            table.at[pl.ds(pl.multiple_of(g * 8, 8), 8),
                     pl.ds(pl.multiple_of(c0 * 128, 128), WW)], buf, sem_in)

    def fire_outs(flat, stride, start, last, j, buf):
        c0 = chunk_col(start, last, j)
        for s in range(8):
            k = g * 8 + s
            pltpu.async_copy(
                buf.at[s, pl.ds(0, WW)],
                flat.at[pl.ds(pl.multiple_of(k * stride + c0 * 128, 128), WW)],
                sem_out)

    def wait_in():
        pltpu.make_async_copy(uembT_hbm.at[pl.ds(0, 8), pl.ds(0, WW)],
                              bufa, sem_in).wait()

    def drain_out():
        pltpu.make_async_copy(uembT_hbm.at[pl.ds(0, 8), pl.ds(0, WW)],
                              bufa, sem_out).wait()

    ustart = si * UPER
    ulast = jnp.minimum(ustart + UPER, UTILES) - W

    fire_in(uembT_hbm, ustart, ulast, 0, bufs[0])
    fire_in(uembT_hbm, ustart, ulast, 1, bufs[1])

    def u_triple(t, _):
        for i in range(3):
            j = t * 3 + i

            @pl.when(j + 2 < UNCH)
            def _():
                @pl.when(j >= 1)
                def _():
                    drain_out()
                fire_in(uembT_hbm, ustart, ulast, j + 2, bufs[(i + 2) % 3])

            wait_in()
            fire_outs(uflat_hbm, USTRIDE, ustart, ulast, j, bufs[i])
        return 0

    lax.fori_loop(0, UNCH // 3, u_triple, 0)
    for j in range((UNCH // 3) * 3, UNCH):
        wait_in()
        fire_outs(uflat_hbm, USTRIDE, ustart, ulast, j, bufs[j % 3])
    drain_out()
    drain_out()
    drain_out()

    vstart = si * VPER
    vlast = jnp.minimum(vstart + VPER, VTILES) - W

    fire_in(vembT_hbm, vstart, vlast, 0, bufs[0])
    fire_in(vembT_hbm, vstart, vlast, 1, bufs[1])
    for j in range(VNCH):
        if j + 2 < VNCH:
            if j >= 1:
                drain_out()
            fire_in(vembT_hbm, vstart, vlast, j + 2, bufs[(j + 2) % 3])
        wait_in()
        fire_outs(vflat_hbm, VSTRIDE, vstart, vlast, j, bufs[j % 3])
    for _ in range(VNCH - max(0, VNCH - 3)):
        drain_out()

    cp1 = pltpu.async_copy(
        utail_hbm.at[pl.ds(wid * 128, 128)],
        uflat_hbm.at[pl.ds(wid * USTRIDE + UMAIN, 128)], sem_in)
    cp2 = pltpu.async_copy(
        vtail_hbm.at[pl.ds(wid * 128, 128)],
        vflat_hbm.at[pl.ds(wid * VSTRIDE + VMAIN, 128)], sem_in)
    cp1.wait()
    cp2.wait()


@functools.partial(
    pl.kernel,
    mesh=_mesh,
    out_type=jax.ShapeDtypeStruct((B,), jnp.float32),
    compiler_params=pltpu.CompilerParams(
        needs_layout_passes=False, use_tc_tiling_on_sc=False),
    scratch_types=[
        pltpu.VMEM((BPW,), jnp.int32),
        pltpu.VMEM((BPW,), jnp.int32),
        pltpu.VMEM((2 * EPW,), jnp.int32),
        pltpu.VMEM((2 * EPW,), jnp.int32),
        pltpu.VMEM((2 * EPW,), jnp.float32),
        pltpu.VMEM((2 * EPW,), jnp.float32),
        pltpu.VMEM((BPW,), jnp.float32),
        pltpu.SemaphoreType.DMA,
    ],
)
def _lookup(uidx_hbm, vidx_hbm, uflat_hbm, vflat_hbm, out_hbm,
            uidx, vidx, uoff, voff, uel, vel, outv, sem):
    wid = lax.axis_index("s") * NC + lax.axis_index("c")
    base = wid * BPW
    pltpu.sync_copy(uidx_hbm.at[pl.ds(base, BPW)], uidx)
    pltpu.sync_copy(vidx_hbm.at[pl.ds(base, BPW)], vidx)

    zeros16 = jnp.zeros((RB,), jnp.float32)
    for b in range(NG):
        outv[pl.ds(b * RB, RB)] = zeros16

    KPWAVE = EPW // BPW

    def dot_wave(w, half):
        for b in range(NG):
            acc = outv[pl.ds(b * RB, RB)]
            for kk in range(KPWAVE):
                acc = acc + (uel[pl.ds(half * EPW + kk * BPW + b * RB, RB)]
                             * vel[pl.ds(half * EPW + kk * BPW + b * RB, RB)])
            outv[pl.ds(b * RB, RB)] = acc

    def wave(w, _):
        half = w % 2
        hoff = half * EPW
        for kk in range(KPWAVE):
            k = w * KPWAVE + kk
            for g in range(NG):
                ru = uidx[pl.ds(g * RB, RB)]
                rv = vidx[pl.ds(g * RB, RB)]
                uoff[pl.ds(hoff + kk * BPW + g * RB, RB)] = ru + k * USTRIDE
                voff[pl.ds(hoff + kk * BPW + g * RB, RB)] = rv + k * VSTRIDE
        for c0 in range(CPW):
            pltpu.async_copy(uflat_hbm.at[uoff.at[pl.ds(hoff + c0 * CH, CH)]],
                             uel.at[pl.ds(hoff + c0 * CH, CH)], sem)
            pltpu.async_copy(vflat_hbm.at[voff.at[pl.ds(hoff + c0 * CH, CH)]],
                             vel.at[pl.ds(hoff + c0 * CH, CH)], sem)

        @pl.when(w > 0)
        def _drain_and_reduce_prev():
            pltpu.make_async_copy(uflat_hbm.at[pl.ds(0, EPW)],
                                  uel.at[pl.ds(0, EPW)], sem).wait()
            pltpu.make_async_copy(vflat_hbm.at[pl.ds(0, EPW)],
                                  vel.at[pl.ds(0, EPW)], sem).wait()
            dot_wave(w - 1, 1 - half)
        return 0

    lax.fori_loop(0, NWAVE, wave, 0)
    pltpu.make_async_copy(uflat_hbm.at[pl.ds(0, EPW)],
                          uel.at[pl.ds(0, EPW)], sem).wait()
    pltpu.make_async_copy(vflat_hbm.at[pl.ds(0, EPW)],
                          vel.at[pl.ds(0, EPW)], sem).wait()
    dot_wave(NWAVE - 1, (NWAVE - 1) % 2)
    pltpu.sync_copy(outv, out_hbm.at[pl.ds(base, BPW)])


def kernel(user_index, item_index, Uemb, Vemb):
    utail = jnp.reshape(
        jnp.pad(Uemb[UMAIN:], ((0, 128 - (UROWS - UMAIN)), (0, 0))).T, (K * 128,))
    vtail = jnp.reshape(
        jnp.pad(Vemb[VMAIN:], ((0, 128 - (VROWS - VMAIN)), (0, 0))).T, (K * 128,))
    uflat, vflat = _relayout(Uemb.T, Vemb.T, utail, vtail)
    return _lookup(user_index.astype(jnp.int32), item_index.astype(jnp.int32),
                   uflat, vflat)
